# Initial kernel scaffold; baseline (speedup 1.0000x reference)
import jax
import jax.numpy as jnp
from jax.experimental import pallas as pl


def kernel(x, table):
    raise NotImplementedError("write your pallas kernel here")


# speedup vs baseline: 1.7781x; 1.7781x over previous
import functools

import jax
import jax.numpy as jnp
from jax import lax
from jax.experimental import pallas as pl
from jax.experimental.pallas import tpu as pltpu
from jax.experimental.pallas import tpu_sc as plsc

VOCAB = 100000
EMB_DIM = 300
PAD_TOKEN = 0
B, NK, L = 1024, 16, 20

NUM_SEG = B * NK
NC, NS = 2, 16
NW = NC * NS
SEG_PER_W = NUM_SEG // NW
S = 4
PITCH = 304
ROWS = S * L
NCHUNK = SEG_PER_W // S

OFFS = tuple(range(0, EMB_DIM - 16, 16)) + (EMB_DIM - 16,)


def _pool_kernel(idx_hbm, table_hbm, out_hbm, idx_v, rows_v, out_v, sem):
    wid = lax.axis_index("s") * NC + lax.axis_index("c")
    wbase = wid * SEG_PER_W

    def chunk_body(c, carry):
        lane = lax.iota(jnp.int32, 16)
        tail_mask = lane >= 12
        seg0 = wbase + c * S
        pltpu.sync_copy(idx_hbm.at[pl.ds(seg0 * L, ROWS)], idx_v)
        pltpu.async_copy(table_hbm.at[idx_v], rows_v, sem).wait()

        for s in range(S):
            base = s * L

            def row_body(r, accs, _base=base):
                row = _base + r
                return tuple(a + rows_v[row, pl.ds(off, 16)]
                             for a, off in zip(accs, OFFS))

            accs = lax.fori_loop(
                0, L, row_body,
                tuple(jnp.zeros((16,), jnp.float32) for _ in OFFS))

            w1 = idx_v[pl.ds(base, 16)]
            w2 = idx_v[pl.ds(base + 4, 16)]
            cnt = (jnp.sum((w1 != PAD_TOKEN).astype(jnp.int32))
                   + jnp.sum(((w2 != PAD_TOKEN) & tail_mask).astype(jnp.int32)))
            cnt_vec = jnp.full((16,), cnt).astype(jnp.float32)

            for a, off in zip(accs, OFFS):
                out_v[s, pl.ds(off, 16)] = a / cnt_vec

        pltpu.sync_copy(out_v, out_hbm.at[pl.ds(seg0, S)])
        return carry

    lax.fori_loop(0, NCHUNK, chunk_body, 0)


@jax.jit
def _emb_pool(x_flat, table):
    mesh = plsc.VectorSubcoreMesh(core_axis_name="c", subcore_axis_name="s")
    fn = functools.partial(
        pl.kernel,
        mesh=mesh,
        out_type=jax.ShapeDtypeStruct((NUM_SEG, EMB_DIM), jnp.float32),
        scratch_types=[
            pltpu.VMEM((ROWS,), jnp.int32),
            pltpu.VMEM((ROWS, PITCH), jnp.float32),
            pltpu.VMEM((S, EMB_DIM), jnp.float32),
            pltpu.SemaphoreType.DMA,
        ],
        compiler_params=pltpu.CompilerParams(
            use_tc_tiling_on_sc=False, needs_layout_passes=False),
    )(_pool_kernel)
    return fn(x_flat, table)


def kernel(x, table):
    out = _emb_pool(x.reshape(-1), jnp.pad(table, ((0, 0), (0, PITCH - EMB_DIM))))
    return out.reshape(B, NK, EMB_DIM)


# speedup vs baseline: 2.2073x; 1.2414x over previous
import functools

import jax
import jax.numpy as jnp
from jax import lax
from jax.experimental import pallas as pl
from jax.experimental.pallas import tpu as pltpu
from jax.experimental.pallas import tpu_sc as plsc

VOCAB = 100000
EMB_DIM = 300
PAD_TOKEN = 0
B, NK, L = 1024, 16, 20

NUM_SEG = B * NK
NC, NS = 2, 16
NW = NC * NS
SEG_PER_W = NUM_SEG // NW
S = 8
PITCH = 384
ROWS = S * L
HALF = ROWS // 2
NCHUNK = SEG_PER_W // S

OFFS = tuple(range(0, EMB_DIM - 16, 16)) + (EMB_DIM - 16,)


def _pool_kernel(idx_hbm, table_hbm, out_hbm, idx_v, rows_v, out_v, sem):
    wid = lax.axis_index("s") * NC + lax.axis_index("c")
    wbase = wid * SEG_PER_W

    def chunk_body(c, carry):
        lane = lax.iota(jnp.int32, 16)
        tail_mask = lane >= 12
        seg0 = wbase + c * S
        pltpu.sync_copy(idx_hbm.at[pl.ds(seg0 * L, ROWS)], idx_v)
        cp0 = pltpu.async_copy(
            table_hbm.at[idx_v.at[pl.ds(0, HALF)]],
            rows_v.at[pl.ds(0, HALF)], sem)
        cp1 = pltpu.async_copy(
            table_hbm.at[idx_v.at[pl.ds(HALF, HALF)]],
            rows_v.at[pl.ds(HALF, HALF)], sem)
        cp0.wait()
        cp1.wait()

        for s in range(S):
            base = s * L

            def row_body(r, accs, _base=base):
                row = _base + r
                return tuple(a + rows_v[row, pl.ds(off, 16)]
                             for a, off in zip(accs, OFFS))

            accs = lax.fori_loop(
                0, L, row_body,
                tuple(jnp.zeros((16,), jnp.float32) for _ in OFFS))

            w1 = idx_v[pl.ds(base, 16)]
            w2 = idx_v[pl.ds(base + 4, 16)]
            cnt = (jnp.sum((w1 != PAD_TOKEN).astype(jnp.int32))
                   + jnp.sum(((w2 != PAD_TOKEN) & tail_mask).astype(jnp.int32)))
            cnt_vec = jnp.full((16,), cnt).astype(jnp.float32)

            for a, off in zip(accs, OFFS):
                out_v[s, pl.ds(off, 16)] = a / cnt_vec

        pltpu.sync_copy(out_v, out_hbm.at[pl.ds(seg0, S)])
        return carry

    lax.fori_loop(0, NCHUNK, chunk_body, 0)


@jax.jit
def _emb_pool(x_flat, table_padded):
    mesh = plsc.VectorSubcoreMesh(core_axis_name="c", subcore_axis_name="s")
    fn = functools.partial(
        pl.kernel,
        mesh=mesh,
        out_type=jax.ShapeDtypeStruct((NUM_SEG, EMB_DIM), jnp.float32),
        scratch_types=[
            pltpu.VMEM((ROWS,), jnp.int32),
            pltpu.VMEM((ROWS, PITCH), jnp.float32),
            pltpu.VMEM((S, EMB_DIM), jnp.float32),
            pltpu.SemaphoreType.DMA,
        ],
        compiler_params=pltpu.CompilerParams(
            use_tc_tiling_on_sc=True, needs_layout_passes=False),
    )(_pool_kernel)
    return fn(x_flat, table_padded)


def kernel(x, table):
    table_padded = jnp.pad(table, ((0, 0), (0, PITCH - EMB_DIM)))
    out = _emb_pool(x.reshape(-1), table_padded)
    return out.reshape(B, NK, EMB_DIM)


# speedup vs baseline: 3.3031x; 1.4964x over previous
import functools

import jax
import jax.numpy as jnp
from jax import lax
from jax.experimental import pallas as pl
from jax.experimental.pallas import tpu as pltpu
from jax.experimental.pallas import tpu_sc as plsc

VOCAB = 100000
EMB_DIM = 300
PAD_TOKEN = 0
B, NK, L = 1024, 16, 20

NUM_SEG = B * NK
NC, NS = 2, 16
NW = NC * NS
SEG_PER_W = NUM_SEG // NW
S = 8
PITCH = 384
ROWS = S * L
HALF = ROWS // 2
NCHUNK = SEG_PER_W // S

OFFS = tuple(range(0, EMB_DIM - 16, 16)) + (EMB_DIM - 16,)


def _pool_kernel(idx_hbm, table_hbm, out_hbm, idx_v, rows_v, out_v, sem):
    wid = lax.axis_index("s") * NC + lax.axis_index("c")
    wbase = wid * SEG_PER_W

    def chunk_body(c, carry):
        lane = lax.iota(jnp.int32, 16)
        tail_mask = lane >= 12
        seg0 = wbase + c * S
        pltpu.sync_copy(idx_hbm.at[pl.ds(seg0 * L, ROWS)], idx_v)
        cp0 = pltpu.async_copy(
            table_hbm.at[idx_v.at[pl.ds(0, HALF)]],
            rows_v.at[pl.ds(0, HALF)], sem)
        cp1 = pltpu.async_copy(
            table_hbm.at[idx_v.at[pl.ds(HALF, HALF)]],
            rows_v.at[pl.ds(HALF, HALF)], sem)
        cp0.wait()
        cp1.wait()

        for s in range(S):
            base = s * L

            def row_body(r, accs, _base=base):
                row = _base + r
                return tuple(a + rows_v[row, pl.ds(off, 16)]
                             for a, off in zip(accs, OFFS))

            accs = lax.fori_loop(
                0, L, row_body,
                tuple(jnp.zeros((16,), jnp.float32) for _ in OFFS))

            w1 = idx_v[pl.ds(base, 16)]
            w2 = idx_v[pl.ds(base + 4, 16)]
            cnt = (jnp.sum((w1 != PAD_TOKEN).astype(jnp.int32))
                   + jnp.sum(((w2 != PAD_TOKEN) & tail_mask).astype(jnp.int32)))
            cnt_vec = jnp.full((16,), cnt).astype(jnp.float32)

            for a, off in zip(accs, OFFS):
                out_v[s, pl.ds(off, 16)] = a / cnt_vec

        pltpu.sync_copy(out_v, out_hbm.at[pl.ds(seg0, S)])
        return carry

    lax.fori_loop(0, NCHUNK, chunk_body, 0)


PAD_BLK = 1000


def _pad_body(x_ref, o_ref):
    o_ref[:, :EMB_DIM] = x_ref[...]


@jax.jit
def _pad_table(table):
    return pl.pallas_call(
        _pad_body,
        grid=(VOCAB // PAD_BLK,),
        in_specs=[pl.BlockSpec((PAD_BLK, EMB_DIM), lambda i: (i, 0))],
        out_specs=pl.BlockSpec((PAD_BLK, PITCH), lambda i: (i, 0)),
        out_shape=jax.ShapeDtypeStruct((VOCAB, PITCH), jnp.float32),
    )(table)


@jax.jit
def _emb_pool(x_flat, table_padded):
    mesh = plsc.VectorSubcoreMesh(core_axis_name="c", subcore_axis_name="s")
    fn = functools.partial(
        pl.kernel,
        mesh=mesh,
        out_type=jax.ShapeDtypeStruct((NUM_SEG, EMB_DIM), jnp.float32),
        scratch_types=[
            pltpu.VMEM((ROWS,), jnp.int32),
            pltpu.VMEM((ROWS, PITCH), jnp.float32),
            pltpu.VMEM((S, EMB_DIM), jnp.float32),
            pltpu.SemaphoreType.DMA,
        ],
        compiler_params=pltpu.CompilerParams(
            use_tc_tiling_on_sc=True, needs_layout_passes=False),
    )(_pool_kernel)
    return fn(x_flat, table_padded)


def kernel(x, table):
    out = _emb_pool(x.reshape(-1), _pad_table(table))
    return out.reshape(B, NK, EMB_DIM)


# speedup vs baseline: 4.1125x; 1.2451x over previous
import functools

import jax
import jax.numpy as jnp
from jax import lax
from jax.experimental import pallas as pl
from jax.experimental.pallas import tpu as pltpu
from jax.experimental.pallas import tpu_sc as plsc

VOCAB = 100000
EMB_DIM = 300
PAD_TOKEN = 0
B, NK, L = 1024, 16, 20

NUM_SEG = B * NK
NC, NS = 2, 16
NW = NC * NS
SEG_PER_W = NUM_SEG // NW
S = 4
PITCH = 384
ROWS = S * L
NCHUNK = SEG_PER_W // S
OUT_STAGE = 64

OFFS = tuple(range(0, EMB_DIM - 16, 16)) + (EMB_DIM - 16,)


def _pool_kernel(idx_hbm, table_hbm, out_hbm, idx2, rows2, out_v,
                 sem0, sem1):
    wid = lax.axis_index("s") * NC + lax.axis_index("c")
    wbase = wid * SEG_PER_W
    sems = (sem0, sem1)

    def copy_idx(g, slot):
        pltpu.sync_copy(idx_hbm.at[pl.ds((wbase + g * S) * L, ROWS)],
                        idx2.at[slot])

    def start_gather(slot):
        pltpu.async_copy(table_hbm.at[idx2.at[slot]], rows2.at[slot],
                         sems[slot])

    def wait_gather(slot):
        pltpu.make_async_copy(table_hbm.at[idx2.at[slot]], rows2.at[slot],
                              sems[slot]).wait()

    def compute_chunk(g2, slot):
        lane = lax.iota(jnp.int32, 16)
        tail_mask = lane >= 12
        stage_base = (g2 % (OUT_STAGE // (2 * S))) * 2 * S + slot * S
        for s in range(S):
            base = s * L

            def row_body(r, accs, _base=base, _slot=slot):
                row = _base + r
                return tuple(a + rows2[_slot, row, pl.ds(off, 16)]
                             for a, off in zip(accs, OFFS))

            accs = lax.fori_loop(
                0, L, row_body,
                tuple(jnp.zeros((16,), jnp.float32) for _ in OFFS))

            w1 = idx2[slot, pl.ds(base, 16)]
            w2 = idx2[slot, pl.ds(base + 4, 16)]
            cnt = (jnp.sum((w1 != PAD_TOKEN).astype(jnp.int32))
                   + jnp.sum(((w2 != PAD_TOKEN) & tail_mask).astype(jnp.int32)))
            cnt_vec = jnp.full((16,), cnt).astype(jnp.float32)

            for a, off in zip(accs, OFFS):
                out_v[stage_base + s, pl.ds(off, 16)] = a / cnt_vec

    copy_idx(0, 0)
    start_gather(0)

    def super_body(g2, carry):
        g = g2 * 2
        copy_idx(g + 1, 1)
        start_gather(1)
        wait_gather(0)
        compute_chunk(g2, 0)
        @pl.when(g2 < NCHUNK // 2 - 1)
        def _():
            copy_idx(g + 2, 0)
            start_gather(0)
        wait_gather(1)
        compute_chunk(g2, 1)

        @pl.when(g2 % (OUT_STAGE // (2 * S)) == OUT_STAGE // (2 * S) - 1)
        def _():
            pltpu.sync_copy(
                out_v,
                out_hbm.at[pl.ds(wbase + (g2 + 1) * 2 * S - OUT_STAGE,
                                 OUT_STAGE)])
        return carry

    lax.fori_loop(0, NCHUNK // 2, super_body, 0)


PAD_BLK = 1000


def _pad_body(x_ref, o_ref):
    o_ref[:, :EMB_DIM] = x_ref[...]


@jax.jit
def _pad_table(table):
    return pl.pallas_call(
        _pad_body,
        grid=(VOCAB // PAD_BLK,),
        in_specs=[pl.BlockSpec((PAD_BLK, EMB_DIM), lambda i: (i, 0))],
        out_specs=pl.BlockSpec((PAD_BLK, PITCH), lambda i: (i, 0)),
        out_shape=jax.ShapeDtypeStruct((VOCAB, PITCH), jnp.float32),
    )(table)


@jax.jit
def _emb_pool(x_flat, table_padded):
    mesh = plsc.VectorSubcoreMesh(core_axis_name="c", subcore_axis_name="s")
    fn = functools.partial(
        pl.kernel,
        mesh=mesh,
        out_type=jax.ShapeDtypeStruct((NUM_SEG, EMB_DIM), jnp.float32),
        scratch_types=[
            pltpu.VMEM((2, ROWS), jnp.int32),
            pltpu.VMEM((2, ROWS, PITCH), jnp.float32),
            pltpu.VMEM((OUT_STAGE, EMB_DIM), jnp.float32),
            pltpu.SemaphoreType.DMA,
            pltpu.SemaphoreType.DMA,
        ],
        compiler_params=pltpu.CompilerParams(
            use_tc_tiling_on_sc=True, needs_layout_passes=False),
    )(_pool_kernel)
    return fn(x_flat, table_padded)


def kernel(x, table):
    out = _emb_pool(x.reshape(-1), _pad_table(table))
    return out.reshape(B, NK, EMB_DIM)


# speedup vs baseline: 4.2621x; 1.0364x over previous
import functools

import jax
import jax.numpy as jnp
from jax import lax
from jax.experimental import pallas as pl
from jax.experimental.pallas import tpu as pltpu
from jax.experimental.pallas import tpu_sc as plsc

VOCAB = 100000
EMB_DIM = 300
PAD_TOKEN = 0
B, NK, L = 1024, 16, 20

NUM_SEG = B * NK
NC, NS = 2, 16
NW = NC * NS
SEG_PER_W = NUM_SEG // NW
S = 4
PITCH = 384
ROWS = S * L
NCHUNK = SEG_PER_W // S
OUT_STAGE = 64

OFFS = tuple(range(0, EMB_DIM - 16, 16)) + (EMB_DIM - 16,)


def _pool_kernel(idx_hbm, table_hbm, out_hbm, idx2, rows2, out_v,
                 sem0, sem1):
    wid = lax.axis_index("s") * NC + lax.axis_index("c")
    wbase = wid * SEG_PER_W
    sems = (sem0, sem1)

    def copy_idx(g, slot):
        pltpu.sync_copy(idx_hbm.at[pl.ds((wbase + g * S) * L, ROWS)],
                        idx2.at[slot])

    def start_gather(slot):
        pltpu.async_copy(table_hbm.at[idx2.at[slot]], rows2.at[slot],
                         sems[slot])

    def wait_gather(slot):
        pltpu.make_async_copy(table_hbm.at[idx2.at[slot]], rows2.at[slot],
                              sems[slot]).wait()

    def compute_chunk(g2, slot):
        lane = lax.iota(jnp.int32, 16)
        tail_mask = lane >= 12
        stage_base = (g2 % (OUT_STAGE // (2 * S))) * 2 * S + slot * S
        for s in range(S):
            base = s * L

            def row_body(r, accs, _base=base, _slot=slot):
                row = _base + r
                return tuple(a + rows2[_slot, row, pl.ds(off, 16)]
                             for a, off in zip(accs, OFFS))

            accs = lax.fori_loop(
                0, L, row_body,
                tuple(jnp.zeros((16,), jnp.float32) for _ in OFFS))

            w1 = idx2[slot, pl.ds(base, 16)]
            w2 = idx2[slot, pl.ds(base + 4, 16)]
            cnt = (jnp.sum((w1 != PAD_TOKEN).astype(jnp.int32))
                   + jnp.sum(((w2 != PAD_TOKEN) & tail_mask).astype(jnp.int32)))
            cnt_vec = jnp.full((16,), cnt).astype(jnp.float32)

            for a, off in zip(accs, OFFS):
                out_v[stage_base + s, pl.ds(off, 16)] = a / cnt_vec

    copy_idx(0, 0)
    start_gather(0)

    def super_body(g2, carry):
        g = g2 * 2
        copy_idx(g + 1, 1)
        start_gather(1)
        wait_gather(0)
        compute_chunk(g2, 0)
        @pl.when(g2 < NCHUNK // 2 - 1)
        def _():
            copy_idx(g + 2, 0)
            start_gather(0)
        wait_gather(1)
        compute_chunk(g2, 1)

        @pl.when(g2 % (OUT_STAGE // (2 * S)) == OUT_STAGE // (2 * S) - 1)
        def _():
            pltpu.sync_copy(
                out_v,
                out_hbm.at[pl.ds(wbase + (g2 + 1) * 2 * S - OUT_STAGE,
                                 OUT_STAGE)])
        return carry

    lax.fori_loop(0, NCHUNK // 2, super_body, 0)


PAD_BLK = 2000


def _pad_body(x_ref, o_ref):
    o_ref[:, :EMB_DIM] = x_ref[...]


def _pad_table(table):
    return pl.pallas_call(
        _pad_body,
        grid=(VOCAB // PAD_BLK,),
        in_specs=[pl.BlockSpec((PAD_BLK, EMB_DIM), lambda i: (i, 0))],
        out_specs=pl.BlockSpec((PAD_BLK, PITCH), lambda i: (i, 0)),
        out_shape=jax.ShapeDtypeStruct((VOCAB, PITCH), jnp.float32),
    )(table)


def _emb_pool(x_flat, table_padded):
    mesh = plsc.VectorSubcoreMesh(core_axis_name="c", subcore_axis_name="s")
    fn = functools.partial(
        pl.kernel,
        mesh=mesh,
        out_type=jax.ShapeDtypeStruct((NUM_SEG, EMB_DIM), jnp.float32),
        scratch_types=[
            pltpu.VMEM((2, ROWS), jnp.int32),
            pltpu.VMEM((2, ROWS, PITCH), jnp.float32),
            pltpu.VMEM((OUT_STAGE, EMB_DIM), jnp.float32),
            pltpu.SemaphoreType.DMA,
            pltpu.SemaphoreType.DMA,
        ],
        compiler_params=pltpu.CompilerParams(
            use_tc_tiling_on_sc=True, needs_layout_passes=False),
    )(_pool_kernel)
    return fn(x_flat, table_padded)


def kernel(x, table):
    out = _emb_pool(x.reshape(-1), _pad_table(table))
    return out.reshape(B, NK, EMB_DIM)
